# Initial kernel scaffold; baseline (speedup 1.0000x reference)
import functools

import jax
import jax.numpy as jnp
from jax import lax
from jax.experimental import pallas as pl
from jax.experimental.pallas import tpu as pltpu
from jax.experimental.pallas import tpu_sc as plsc

NC = 2
NS = 16
K = 80

_MESH = plsc.VectorSubcoreMesh(core_axis_name="c", subcore_axis_name="s")


def _sc_degree(idx2d, zeros_n, n):
    nrows = idx2d.shape[0]
    rows_per_worker = nrows // (NC * NS)

    @functools.partial(
        pl.kernel,
        out_type=jax.ShapeDtypeStruct((2, n), jnp.float32),
        mesh=_MESH,
        scratch_types=[
            pltpu.VMEM_SHARED((n,), jnp.float32),
            pltpu.VMEM((rows_per_worker, K), jnp.int32),
            pltpu.VMEM((K,), jnp.float32),
            pltpu.VMEM((n,), jnp.float32),
        ],
    )
    def deg_kernel(idx_hbm, z_hbm, out_hbm, acc, idxb, ones, stage):
        c = lax.axis_index("c")
        s = lax.axis_index("s")

        def set_ones(i, _):
            ones[pl.ds(i * 16, 16)] = jnp.full((16,), 1.0, jnp.float32)
            return 0
        lax.fori_loop(0, K // 16, set_ones, 0)

        @pl.when(s == 0)
        def _():
            pltpu.sync_copy(z_hbm, stage)
            pltpu.sync_copy(stage, acc)
        plsc.subcore_barrier()

        rowbase = (c * NS + s) * rows_per_worker
        pltpu.sync_copy(idx_hbm.at[pl.ds(rowbase, rows_per_worker)], idxb)

        def body(j, _):
            pltpu.sync_copy(ones, acc.at[idxb.at[j]], add=True)
            return 0
        lax.fori_loop(0, rows_per_worker, body, 0)
        plsc.subcore_barrier()

        @pl.when(s == 0)
        def _():
            pltpu.sync_copy(acc, stage)
            pltpu.sync_copy(stage, out_hbm.at[c])

    return deg_kernel(idx2d, zeros_n)


def _sc_edge_agg(hn, src2d, dst2d, zeros_nd, n, d):
    nrows = src2d.shape[0]
    rows_per_worker = nrows // (NC * NS)
    rows_n = n // NS

    @functools.partial(
        pl.kernel,
        out_type=jax.ShapeDtypeStruct((2 * n, d), jnp.float32),
        mesh=_MESH,
        scratch_types=[
            pltpu.VMEM_SHARED((n, d), jnp.float32),
            pltpu.VMEM((rows_per_worker, K), jnp.int32),
            pltpu.VMEM((rows_per_worker, K), jnp.int32),
            pltpu.VMEM((K, d), jnp.float32),
            pltpu.VMEM((rows_n, d), jnp.float32),
        ],
    )
    def edge_kernel(hn_hbm, src_hbm, dst_hbm, z_hbm, out_hbm,
                    acc, srcb, dstb, rows, stage):
        c = lax.axis_index("c")
        s = lax.axis_index("s")

        accrows = acc.at[pl.ds(s * rows_n, rows_n)]
        pltpu.sync_copy(z_hbm.at[pl.ds(s * rows_n, rows_n)], stage)
        pltpu.sync_copy(stage, accrows)
        plsc.subcore_barrier()

        rowbase = (c * NS + s) * rows_per_worker
        pltpu.sync_copy(src_hbm.at[pl.ds(rowbase, rows_per_worker)], srcb)
        pltpu.sync_copy(dst_hbm.at[pl.ds(rowbase, rows_per_worker)], dstb)

        def body(j, _):
            pltpu.sync_copy(hn_hbm.at[srcb.at[j]], rows)
            pltpu.sync_copy(rows, acc.at[dstb.at[j]], add=True)
            return 0
        lax.fori_loop(0, rows_per_worker, body, 0)
        plsc.subcore_barrier()

        pltpu.sync_copy(accrows, stage)
        pltpu.sync_copy(stage, out_hbm.at[pl.ds(c * n + s * rows_n, rows_n)])

    return edge_kernel(hn, src2d, dst2d, zeros_nd)


def _tc_prep(h, cnt_nc, n, d, blk):
    def prep_kernel(h_ref, c_ref, hn_ref, sc_ref):
        s = lax.rsqrt(jnp.maximum(c_ref[...], 1.0))
        sc_ref[...] = s
        hn_ref[...] = h_ref[...] * s[:, 0:1]

    return pl.pallas_call(
        prep_kernel,
        grid=(n // blk,),
        in_specs=[
            pl.BlockSpec((blk, d), lambda i: (i, 0)),
            pl.BlockSpec((blk, 2), lambda i: (i, 0)),
        ],
        out_specs=[
            pl.BlockSpec((blk, d), lambda i: (i, 0)),
            pl.BlockSpec((blk, 2), lambda i: (i, 0)),
        ],
        out_shape=[
            jax.ShapeDtypeStruct((n, d), jnp.float32),
            jax.ShapeDtypeStruct((n, 2), jnp.float32),
        ],
    )(h, cnt_nc)


def _tc_layer(p0, p1, scol, W, b2d, n, d, blk, norm_out):
    def layer_kernel(p0_ref, p1_ref, sc_ref, w_ref, b_ref, o_ref):
        agg = (p0_ref[...] + p1_ref[...]) * sc_ref[:, 1:2]
        z = jnp.dot(agg, w_ref[...], precision=lax.Precision.HIGHEST,
                    preferred_element_type=jnp.float32)
        hv = jnp.maximum(z + b_ref[...], 0.0)
        if norm_out:
            hv = hv * sc_ref[:, 0:1]
        o_ref[...] = hv

    return pl.pallas_call(
        layer_kernel,
        grid=(n // blk,),
        in_specs=[
            pl.BlockSpec((blk, d), lambda i: (i, 0)),
            pl.BlockSpec((blk, d), lambda i: (i, 0)),
            pl.BlockSpec((blk, 2), lambda i: (i, 0)),
            pl.BlockSpec((d, d), lambda i: (0, 0)),
            pl.BlockSpec((1, d), lambda i: (0, 0)),
        ],
        out_specs=pl.BlockSpec((blk, d), lambda i: (i, 0)),
        out_shape=jax.ShapeDtypeStruct((n, d), jnp.float32),
    )(p0, p1, scol, W, b2d)


def kernel(h, edge_index, W1, b1, W2, b2):
    n, d = h.shape
    e = edge_index.shape[1]
    blk = 1000

    idx2d = edge_index.reshape(2 * e // K, K)
    src2d = edge_index[0].reshape(e // K, K)
    dst2d = edge_index[1].reshape(e // K, K)
    zeros_n = jnp.zeros((n,), jnp.float32)
    zeros_nd = jnp.zeros((n, d), jnp.float32)

    cnt = _sc_degree(idx2d, zeros_n, n)
    cnt_nc = cnt.T
    hn0, scol = _tc_prep(h, cnt_nc, n, d, blk)

    part1 = _sc_edge_agg(hn0, src2d, dst2d, zeros_nd, n, d)
    h1n = _tc_layer(part1[:n], part1[n:], scol, W1, b1.reshape(1, d),
                    n, d, blk, norm_out=True)

    part2 = _sc_edge_agg(h1n, src2d, dst2d, zeros_nd, n, d)
    out = _tc_layer(part2[:n], part2[n:], scol, W2, b2.reshape(1, d),
                    n, d, blk, norm_out=False)
    return out


# speedup vs baseline: 8.7130x; 8.7130x over previous
import functools

import jax
import jax.numpy as jnp
from jax import lax
from jax.experimental import pallas as pl
from jax.experimental.pallas import tpu as pltpu
from jax.experimental.pallas import tpu_sc as plsc

NC = 2
NS = 16
K = 125
ZR = 624

_MESH = plsc.VectorSubcoreMesh(core_axis_name="c", subcore_axis_name="s")


def _sc_degree(idx2d, zeros_n, n):
    nrows = idx2d.shape[0]
    rpw = nrows // (NC * NS)

    @functools.partial(
        pl.kernel,
        out_type=jax.ShapeDtypeStruct((2, n), jnp.float32),
        mesh=_MESH,
        compiler_params=pltpu.CompilerParams(use_tc_tiling_on_sc=False),
        scratch_types=[
            pltpu.VMEM_SHARED((n,), jnp.float32),
            pltpu.VMEM((rpw, K), jnp.int32),
            pltpu.VMEM((128,), jnp.float32),
            pltpu.VMEM((n,), jnp.float32),
        ],
    )
    def deg_kernel(idx_hbm, z_hbm, out_hbm, acc, idxb, ones, stage):
        c = lax.axis_index("c")
        s = lax.axis_index("s")

        def set_ones(i, _):
            ones[pl.ds(i * 16, 16)] = jnp.full((16,), 1.0, jnp.float32)
            return 0
        lax.fori_loop(0, 8, set_ones, 0)

        @pl.when(s == 0)
        def _():
            pltpu.sync_copy(z_hbm, stage)
            pltpu.sync_copy(stage, acc)
        plsc.subcore_barrier()

        rowbase = (c * NS + s) * rpw
        pltpu.sync_copy(idx_hbm.at[pl.ds(rowbase, rpw)], idxb)

        def body(j, _):
            pltpu.sync_copy(ones.at[pl.ds(0, K)], acc.at[idxb.at[j]], add=True)
            return 0
        lax.fori_loop(0, rpw, body, 0)
        plsc.subcore_barrier()

        @pl.when(s == 0)
        def _():
            pltpu.sync_copy(acc, stage)
            pltpu.sync_copy(stage, out_hbm.at[c])

    return deg_kernel(idx2d, zeros_n)


def _sc_edge_agg(hn, src2d, dst2d, zeros_nd, n, d):
    nrows = src2d.shape[0]
    rpw = nrows // (NC * NS)
    zlast = n - (NS - 1) * ZR

    @functools.partial(
        pl.kernel,
        out_type=jax.ShapeDtypeStruct((2 * n, d), jnp.float32),
        mesh=_MESH,
        compiler_params=pltpu.CompilerParams(use_tc_tiling_on_sc=False),
        scratch_types=[
            pltpu.VMEM_SHARED((n, d), jnp.float32),
            pltpu.VMEM((rpw, K), jnp.int32),
            pltpu.VMEM((rpw, K), jnp.int32),
            pltpu.VMEM((K, d), jnp.float32),
        ],
    )
    def edge_kernel(hn_hbm, src_hbm, dst_hbm, z_hbm, out_hbm,
                    acc, srcb, dstb, rows):
        c = lax.axis_index("c")
        s = lax.axis_index("s")

        @pl.when(s < NS - 1)
        def _():
            pltpu.sync_copy(z_hbm.at[pl.ds(s * ZR, ZR)],
                            acc.at[pl.ds(s * ZR, ZR)])

        @pl.when(s == NS - 1)
        def _():
            pltpu.sync_copy(z_hbm.at[pl.ds((NS - 1) * ZR, zlast)],
                            acc.at[pl.ds((NS - 1) * ZR, zlast)])
        plsc.subcore_barrier()

        rowbase = (c * NS + s) * rpw
        pltpu.sync_copy(src_hbm.at[pl.ds(rowbase, rpw)], srcb)
        pltpu.sync_copy(dst_hbm.at[pl.ds(rowbase, rpw)], dstb)

        def body(j, _):
            pltpu.sync_copy(hn_hbm.at[srcb.at[j]], rows)
            pltpu.sync_copy(rows, acc.at[dstb.at[j]], add=True)
            return 0
        lax.fori_loop(0, rpw, body, 0)
        plsc.subcore_barrier()

        @pl.when(s < NS - 1)
        def _():
            pltpu.sync_copy(acc.at[pl.ds(s * ZR, ZR)],
                            out_hbm.at[pl.ds(c * n + s * ZR, ZR)])

        @pl.when(s == NS - 1)
        def _():
            pltpu.sync_copy(acc.at[pl.ds((NS - 1) * ZR, zlast)],
                            out_hbm.at[pl.ds(c * n + (NS - 1) * ZR, zlast)])

    return edge_kernel(hn, src2d, dst2d, zeros_nd)


def _tc_prep(h, cnt_nc, n, d, blk):
    def prep_kernel(h_ref, c_ref, hn_ref, sc_ref):
        s = lax.rsqrt(jnp.maximum(c_ref[...], 1.0))
        sc_ref[...] = s
        hn_ref[...] = h_ref[...] * s[:, 0:1]

    return pl.pallas_call(
        prep_kernel,
        grid=(n // blk,),
        in_specs=[
            pl.BlockSpec((blk, d), lambda i: (i, 0)),
            pl.BlockSpec((blk, 2), lambda i: (i, 0)),
        ],
        out_specs=[
            pl.BlockSpec((blk, d), lambda i: (i, 0)),
            pl.BlockSpec((blk, 2), lambda i: (i, 0)),
        ],
        out_shape=[
            jax.ShapeDtypeStruct((n, d), jnp.float32),
            jax.ShapeDtypeStruct((n, 2), jnp.float32),
        ],
    )(h, cnt_nc)


def _tc_layer(p0, p1, scol, W, b2d, n, d, blk, norm_out):
    def layer_kernel(p0_ref, p1_ref, sc_ref, w_ref, b_ref, o_ref):
        agg = (p0_ref[...] + p1_ref[...]) * sc_ref[:, 1:2]
        z = jnp.dot(agg, w_ref[...], precision=lax.Precision.HIGHEST,
                    preferred_element_type=jnp.float32)
        hv = jnp.maximum(z + b_ref[...], 0.0)
        if norm_out:
            hv = hv * sc_ref[:, 0:1]
        o_ref[...] = hv

    return pl.pallas_call(
        layer_kernel,
        grid=(n // blk,),
        in_specs=[
            pl.BlockSpec((blk, d), lambda i: (i, 0)),
            pl.BlockSpec((blk, d), lambda i: (i, 0)),
            pl.BlockSpec((blk, 2), lambda i: (i, 0)),
            pl.BlockSpec((d, d), lambda i: (0, 0)),
            pl.BlockSpec((1, d), lambda i: (0, 0)),
        ],
        out_specs=pl.BlockSpec((blk, d), lambda i: (i, 0)),
        out_shape=jax.ShapeDtypeStruct((n, d), jnp.float32),
    )(p0, p1, scol, W, b2d)


def kernel(h, edge_index, W1, b1, W2, b2):
    n, d = h.shape
    e = edge_index.shape[1]
    blk = 1000

    idx2d = edge_index.reshape(2 * e // K, K)
    src2d = edge_index[0].reshape(e // K, K)
    dst2d = edge_index[1].reshape(e // K, K)
    zeros_n = jnp.zeros((n,), jnp.float32)
    zeros_nd = jnp.zeros((n, d), jnp.float32)

    cnt = _sc_degree(idx2d, zeros_n, n)
    cnt_nc = cnt.T
    hn0, scol = _tc_prep(h, cnt_nc, n, d, blk)

    part1 = _sc_edge_agg(hn0, src2d, dst2d, zeros_nd, n, d)
    h1n = _tc_layer(part1[:n], part1[n:], scol, W1, b1.reshape(1, d),
                    n, d, blk, norm_out=True)

    part2 = _sc_edge_agg(h1n, src2d, dst2d, zeros_nd, n, d)
    out = _tc_layer(part2[:n], part2[n:], scol, W2, b2.reshape(1, d),
                    n, d, blk, norm_out=False)
    return out


# speedup vs baseline: 12.0349x; 1.3813x over previous
import functools

import jax
import jax.numpy as jnp
from jax import lax
from jax.experimental import pallas as pl
from jax.experimental.pallas import tpu as pltpu
from jax.experimental.pallas import tpu_sc as plsc

NC = 2
NS = 16
K = 50
ZR = 624

_MESH = plsc.VectorSubcoreMesh(core_axis_name="c", subcore_axis_name="s")


def _sc_degree(idx2d, zeros_n, n):
    nrows = idx2d.shape[0]
    rpw = nrows // (NC * NS)

    @functools.partial(
        pl.kernel,
        out_type=jax.ShapeDtypeStruct((2, n), jnp.float32),
        mesh=_MESH,
        compiler_params=pltpu.CompilerParams(use_tc_tiling_on_sc=False),
        scratch_types=[
            pltpu.VMEM_SHARED((n,), jnp.float32),
            pltpu.VMEM((rpw, K), jnp.int32),
            pltpu.VMEM((128,), jnp.float32),
            pltpu.VMEM((n,), jnp.float32),
            pltpu.SemaphoreType.DMA,
        ],
    )
    def deg_kernel(idx_hbm, z_hbm, out_hbm, acc, idxb, ones, stage, ssem):
        c = lax.axis_index("c")
        s = lax.axis_index("s")

        def set_ones(i, _):
            ones[pl.ds(i * 16, 16)] = jnp.full((16,), 1.0, jnp.float32)
            return 0
        lax.fori_loop(0, 8, set_ones, 0)

        @pl.when(s == 0)
        def _():
            pltpu.sync_copy(z_hbm, stage)
            pltpu.sync_copy(stage, acc)
        plsc.subcore_barrier()

        rowbase = (c * NS + s) * rpw
        pltpu.sync_copy(idx_hbm.at[pl.ds(rowbase, rpw)], idxb)

        W = 4

        def body(j, _):
            @pl.when(j >= W)
            def _():
                pltpu.make_async_copy(
                    ones.at[pl.ds(0, K)], acc.at[idxb.at[j]], ssem).wait()
            pltpu.async_copy(
                ones.at[pl.ds(0, K)], acc.at[idxb.at[j]], ssem, add=True)
            return 0
        lax.fori_loop(0, rpw, body, 0)

        def drain(j, _):
            pltpu.make_async_copy(
                ones.at[pl.ds(0, K)], acc.at[idxb.at[j]], ssem).wait()
            return 0
        lax.fori_loop(0, W, drain, 0)
        plsc.subcore_barrier()

        @pl.when(s == 0)
        def _():
            pltpu.sync_copy(acc, stage)
            pltpu.sync_copy(stage, out_hbm.at[c])

    return deg_kernel(idx2d, zeros_n)


def _sc_edge_agg(hn, src2d, dst2d, zeros_nd, n, d):
    nrows = src2d.shape[0]
    rpw = nrows // (NC * NS)
    zlast = n - (NS - 1) * ZR

    @functools.partial(
        pl.kernel,
        out_type=jax.ShapeDtypeStruct((2 * n, d), jnp.float32),
        mesh=_MESH,
        compiler_params=pltpu.CompilerParams(use_tc_tiling_on_sc=False),
        scratch_types=[
            pltpu.VMEM_SHARED((n, d), jnp.float32),
            pltpu.VMEM((rpw, K), jnp.int32),
            pltpu.VMEM((rpw, K), jnp.int32),
            pltpu.VMEM((K, d), jnp.float32),
            pltpu.VMEM((K, d), jnp.float32),
            pltpu.VMEM((K, d), jnp.float32),
            pltpu.VMEM((K, d), jnp.float32),
            pltpu.SemaphoreType.DMA,
            pltpu.SemaphoreType.DMA,
            pltpu.SemaphoreType.DMA,
            pltpu.SemaphoreType.DMA,
            pltpu.SemaphoreType.DMA,
            pltpu.SemaphoreType.DMA,
            pltpu.SemaphoreType.DMA,
            pltpu.SemaphoreType.DMA,
        ],
    )
    def edge_kernel(hn_hbm, src_hbm, dst_hbm, z_hbm, out_hbm,
                    acc, srcb, dstb, r0, r1, r2, r3,
                    g0, g1, g2, g3, s0, s1, s2, s3):
        rows = (r0, r1, r2, r3)
        gsem = (g0, g1, g2, g3)
        ssem = (s0, s1, s2, s3)
        c = lax.axis_index("c")
        s = lax.axis_index("s")

        @pl.when(s < NS - 1)
        def _():
            pltpu.sync_copy(z_hbm.at[pl.ds(s * ZR, ZR)],
                            acc.at[pl.ds(s * ZR, ZR)])

        @pl.when(s == NS - 1)
        def _():
            pltpu.sync_copy(z_hbm.at[pl.ds((NS - 1) * ZR, zlast)],
                            acc.at[pl.ds((NS - 1) * ZR, zlast)])
        plsc.subcore_barrier()

        rowbase = (c * NS + s) * rpw
        pltpu.sync_copy(src_hbm.at[pl.ds(rowbase, rpw)], srcb)
        pltpu.sync_copy(dst_hbm.at[pl.ds(rowbase, rpw)], dstb)

        ngrp = rpw // 4
        for b in range(4):
            pltpu.async_copy(hn_hbm.at[srcb.at[b]], rows[b], gsem[b])

        def group(jj, _):
            for b in range(4):
                j = jj * 4 + b
                pltpu.make_async_copy(
                    hn_hbm.at[srcb.at[j]], rows[b], gsem[b]).wait()
                pltpu.async_copy(rows[b], acc.at[dstb.at[j]], ssem[b],
                                 add=True)

                @pl.when(jj < ngrp - 1)
                def _():
                    pltpu.make_async_copy(
                        rows[b], acc.at[dstb.at[j]], ssem[b]).wait()
                    pltpu.async_copy(
                        hn_hbm.at[srcb.at[j + 4]], rows[b], gsem[b])
            return 0
        lax.fori_loop(0, ngrp, group, 0)

        for b in range(4):
            pltpu.make_async_copy(
                rows[b], acc.at[dstb.at[rpw - 4 + b]], ssem[b]).wait()
        plsc.subcore_barrier()

        @pl.when(s < NS - 1)
        def _():
            pltpu.sync_copy(acc.at[pl.ds(s * ZR, ZR)],
                            out_hbm.at[pl.ds(c * n + s * ZR, ZR)])

        @pl.when(s == NS - 1)
        def _():
            pltpu.sync_copy(acc.at[pl.ds((NS - 1) * ZR, zlast)],
                            out_hbm.at[pl.ds(c * n + (NS - 1) * ZR, zlast)])

    return edge_kernel(hn, src2d, dst2d, zeros_nd)


def _tc_prep(h, cnt_nc, n, d, blk):
    def prep_kernel(h_ref, c_ref, hn_ref, sc_ref):
        s = lax.rsqrt(jnp.maximum(c_ref[...], 1.0))
        sc_ref[...] = s
        hn_ref[...] = h_ref[...] * s[:, 0:1]

    return pl.pallas_call(
        prep_kernel,
        grid=(n // blk,),
        in_specs=[
            pl.BlockSpec((blk, d), lambda i: (i, 0)),
            pl.BlockSpec((blk, 2), lambda i: (i, 0)),
        ],
        out_specs=[
            pl.BlockSpec((blk, d), lambda i: (i, 0)),
            pl.BlockSpec((blk, 2), lambda i: (i, 0)),
        ],
        out_shape=[
            jax.ShapeDtypeStruct((n, d), jnp.float32),
            jax.ShapeDtypeStruct((n, 2), jnp.float32),
        ],
    )(h, cnt_nc)


def _tc_layer(p0, p1, scol, W, b2d, n, d, blk, norm_out):
    def layer_kernel(p0_ref, p1_ref, sc_ref, w_ref, b_ref, o_ref):
        agg = (p0_ref[...] + p1_ref[...]) * sc_ref[:, 1:2]
        z = jnp.dot(agg, w_ref[...], precision=lax.Precision.HIGHEST,
                    preferred_element_type=jnp.float32)
        hv = jnp.maximum(z + b_ref[...], 0.0)
        if norm_out:
            hv = hv * sc_ref[:, 0:1]
        o_ref[...] = hv

    return pl.pallas_call(
        layer_kernel,
        grid=(n // blk,),
        in_specs=[
            pl.BlockSpec((blk, d), lambda i: (i, 0)),
            pl.BlockSpec((blk, d), lambda i: (i, 0)),
            pl.BlockSpec((blk, 2), lambda i: (i, 0)),
            pl.BlockSpec((d, d), lambda i: (0, 0)),
            pl.BlockSpec((1, d), lambda i: (0, 0)),
        ],
        out_specs=pl.BlockSpec((blk, d), lambda i: (i, 0)),
        out_shape=jax.ShapeDtypeStruct((n, d), jnp.float32),
    )(p0, p1, scol, W, b2d)


def kernel(h, edge_index, W1, b1, W2, b2):
    n, d = h.shape
    e = edge_index.shape[1]
    blk = 1000

    idx2d = edge_index.reshape(2 * e // K, K)
    src2d = edge_index[0].reshape(e // K, K)
    dst2d = edge_index[1].reshape(e // K, K)
    zeros_n = jnp.zeros((n,), jnp.float32)
    zeros_nd = jnp.zeros((n, d), jnp.float32)

    cnt = _sc_degree(idx2d, zeros_n, n)
    cnt_nc = cnt.T
    hn0, scol = _tc_prep(h, cnt_nc, n, d, blk)

    part1 = _sc_edge_agg(hn0, src2d, dst2d, zeros_nd, n, d)
    h1n = _tc_layer(part1[:n], part1[n:], scol, W1, b1.reshape(1, d),
                    n, d, blk, norm_out=True)

    part2 = _sc_edge_agg(h1n, src2d, dst2d, zeros_nd, n, d)
    out = _tc_layer(part2[:n], part2[n:], scol, W2, b2.reshape(1, d),
                    n, d, blk, norm_out=False)
    return out


# speedup vs baseline: 12.5468x; 1.0425x over previous
import functools

import jax
import jax.numpy as jnp
from jax import lax
from jax.experimental import pallas as pl
from jax.experimental.pallas import tpu as pltpu
from jax.experimental.pallas import tpu_sc as plsc

NC = 2
NS = 16
K = 50
ZR = 624

_MESH = plsc.VectorSubcoreMesh(core_axis_name="c", subcore_axis_name="s")


def _sc_degree(idx2d, zeros_n, n):
    nrows = idx2d.shape[0]
    rpw = nrows // (NC * NS)

    @functools.partial(
        pl.kernel,
        out_type=jax.ShapeDtypeStruct((2, n), jnp.float32),
        mesh=_MESH,
        compiler_params=pltpu.CompilerParams(use_tc_tiling_on_sc=False),
        scratch_types=[
            pltpu.VMEM_SHARED((n,), jnp.float32),
            pltpu.VMEM((rpw, K), jnp.int32),
            pltpu.VMEM((128,), jnp.float32),
            pltpu.VMEM((n,), jnp.float32),
            pltpu.SemaphoreType.DMA,
        ],
    )
    def deg_kernel(idx_hbm, z_hbm, out_hbm, acc, idxb, ones, stage, ssem):
        c = lax.axis_index("c")
        s = lax.axis_index("s")

        def set_ones(i, _):
            ones[pl.ds(i * 16, 16)] = jnp.full((16,), 1.0, jnp.float32)
            return 0
        lax.fori_loop(0, 8, set_ones, 0)

        @pl.when(s == 0)
        def _():
            pltpu.sync_copy(z_hbm, stage)
            pltpu.sync_copy(stage, acc)
        plsc.subcore_barrier()

        rowbase = (c * NS + s) * rpw
        pltpu.sync_copy(idx_hbm.at[pl.ds(rowbase, rpw)], idxb)

        W = 4

        def body(j, _):
            @pl.when(j >= W)
            def _():
                pltpu.make_async_copy(
                    ones.at[pl.ds(0, K)], acc.at[idxb.at[j]], ssem).wait()
            pltpu.async_copy(
                ones.at[pl.ds(0, K)], acc.at[idxb.at[j]], ssem, add=True)
            return 0
        lax.fori_loop(0, rpw, body, 0)

        def drain(j, _):
            pltpu.make_async_copy(
                ones.at[pl.ds(0, K)], acc.at[idxb.at[j]], ssem).wait()
            return 0
        lax.fori_loop(0, W, drain, 0)
        plsc.subcore_barrier()

        @pl.when(s == 0)
        def _():
            pltpu.sync_copy(acc, stage)
            pltpu.sync_copy(stage, out_hbm.at[c])

    return deg_kernel(idx2d, zeros_n)


def _sc_edge_agg(hn, src2d, dst2d, zeros_nd, n, d):
    nrows = src2d.shape[0]
    rpw = nrows // (NC * NS)
    zlast = n - (NS - 1) * ZR

    @functools.partial(
        pl.kernel,
        out_type=jax.ShapeDtypeStruct((2 * n, d), jnp.float32),
        mesh=_MESH,
        compiler_params=pltpu.CompilerParams(use_tc_tiling_on_sc=False),
        scratch_types=[
            pltpu.VMEM_SHARED((n, d), jnp.float32),
            pltpu.VMEM((rpw, K), jnp.int32),
            pltpu.VMEM((rpw, K), jnp.int32),
            pltpu.VMEM((K, d), jnp.float32),
            pltpu.VMEM((K, d), jnp.float32),
            pltpu.VMEM((K, d), jnp.float32),
            pltpu.VMEM((K, d), jnp.float32),
            pltpu.SemaphoreType.DMA,
            pltpu.SemaphoreType.DMA,
            pltpu.SemaphoreType.DMA,
            pltpu.SemaphoreType.DMA,
            pltpu.SemaphoreType.DMA,
            pltpu.SemaphoreType.DMA,
            pltpu.SemaphoreType.DMA,
            pltpu.SemaphoreType.DMA,
        ],
    )
    def edge_kernel(hn_hbm, src_hbm, dst_hbm, z_hbm, out_hbm,
                    acc, srcb, dstb, r0, r1, r2, r3,
                    g0, g1, g2, g3, s0, s1, s2, s3):
        rows = (r0, r1, r2, r3)
        gsem = (g0, g1, g2, g3)
        ssem = (s0, s1, s2, s3)
        c = lax.axis_index("c")
        s = lax.axis_index("s")

        @pl.when(s < NS - 1)
        def _():
            pltpu.sync_copy(z_hbm.at[pl.ds(s * ZR, ZR)],
                            acc.at[pl.ds(s * ZR, ZR)])

        @pl.when(s == NS - 1)
        def _():
            pltpu.sync_copy(z_hbm.at[pl.ds((NS - 1) * ZR, zlast)],
                            acc.at[pl.ds((NS - 1) * ZR, zlast)])
        plsc.subcore_barrier()

        rowbase = (c * NS + s) * rpw
        pltpu.sync_copy(src_hbm.at[pl.ds(rowbase, rpw)], srcb)
        pltpu.sync_copy(dst_hbm.at[pl.ds(rowbase, rpw)], dstb)

        ngrp = rpw // 4
        for b in range(4):
            pltpu.async_copy(hn_hbm.at[srcb.at[b]], rows[b], gsem[b])

        def group(jj, _):
            for b in range(4):
                j = jj * 4 + b
                pltpu.make_async_copy(
                    hn_hbm.at[srcb.at[j]], rows[b], gsem[b]).wait()
                pltpu.async_copy(rows[b], acc.at[dstb.at[j]], ssem[b],
                                 add=True)

                @pl.when(jj < ngrp - 1)
                def _():
                    pltpu.make_async_copy(
                        rows[b], acc.at[dstb.at[j]], ssem[b]).wait()
                    pltpu.async_copy(
                        hn_hbm.at[srcb.at[j + 4]], rows[b], gsem[b])
            return 0
        lax.fori_loop(0, ngrp, group, 0)

        for b in range(4):
            pltpu.make_async_copy(
                rows[b], acc.at[dstb.at[rpw - 4 + b]], ssem[b]).wait()
        plsc.subcore_barrier()

        @pl.when(s < NS - 1)
        def _():
            pltpu.sync_copy(acc.at[pl.ds(s * ZR, ZR)],
                            out_hbm.at[pl.ds(c * n + s * ZR, ZR)])

        @pl.when(s == NS - 1)
        def _():
            pltpu.sync_copy(acc.at[pl.ds((NS - 1) * ZR, zlast)],
                            out_hbm.at[pl.ds(c * n + (NS - 1) * ZR, zlast)])

    return edge_kernel(hn, src2d, dst2d, zeros_nd)


def _tc_prep(h, cnt_nc, n, d, blk):
    def prep_kernel(h_ref, c_ref, hn_ref, sc_ref):
        s = lax.rsqrt(jnp.maximum(c_ref[...], 1.0))
        sc_ref[...] = s
        hn_ref[...] = h_ref[...] * s[:, 0:1]

    return pl.pallas_call(
        prep_kernel,
        grid=(n // blk,),
        in_specs=[
            pl.BlockSpec((blk, d), lambda i: (i, 0)),
            pl.BlockSpec((blk, 2), lambda i: (i, 0)),
        ],
        out_specs=[
            pl.BlockSpec((blk, d), lambda i: (i, 0)),
            pl.BlockSpec((blk, 2), lambda i: (i, 0)),
        ],
        out_shape=[
            jax.ShapeDtypeStruct((n, d), jnp.float32),
            jax.ShapeDtypeStruct((n, 2), jnp.float32),
        ],
    )(h, cnt_nc)


def _tc_layer(part, scol, W, b2d, n, d, blk, norm_out):
    nb = n // blk

    def layer_kernel(p0_ref, p1_ref, sc_ref, w_ref, b_ref, o_ref):
        agg = (p0_ref[...] + p1_ref[...]) * sc_ref[:, 1:2]
        z = jnp.dot(agg, w_ref[...], precision=lax.Precision.HIGHEST,
                    preferred_element_type=jnp.float32)
        hv = jnp.maximum(z + b_ref[...], 0.0)
        if norm_out:
            hv = hv * sc_ref[:, 0:1]
        o_ref[...] = hv

    return pl.pallas_call(
        layer_kernel,
        grid=(nb,),
        in_specs=[
            pl.BlockSpec((blk, d), lambda i: (i, 0)),
            pl.BlockSpec((blk, d), lambda i: (i + nb, 0)),
            pl.BlockSpec((blk, 2), lambda i: (i, 0)),
            pl.BlockSpec((d, d), lambda i: (0, 0)),
            pl.BlockSpec((1, d), lambda i: (0, 0)),
        ],
        out_specs=pl.BlockSpec((blk, d), lambda i: (i, 0)),
        out_shape=jax.ShapeDtypeStruct((n, d), jnp.float32),
    )(part, part, scol, W, b2d)


def kernel(h, edge_index, W1, b1, W2, b2):
    n, d = h.shape
    e = edge_index.shape[1]
    blk = 1000

    idx2d = edge_index.reshape(2 * e // K, K)
    src2d = edge_index[0].reshape(e // K, K)
    dst2d = edge_index[1].reshape(e // K, K)
    zeros_n = jnp.zeros((n,), jnp.float32)
    zeros_nd = jnp.zeros((n, d), jnp.float32)

    cnt = _sc_degree(idx2d, zeros_n, n)
    cnt_nc = cnt.T
    hn0, scol = _tc_prep(h, cnt_nc, n, d, blk)

    part1 = _sc_edge_agg(hn0, src2d, dst2d, zeros_nd, n, d)
    h1n = _tc_layer(part1, scol, W1, b1.reshape(1, d),
                    n, d, blk, norm_out=True)

    part2 = _sc_edge_agg(h1n, src2d, dst2d, zeros_nd, n, d)
    out = _tc_layer(part2, scol, W2, b2.reshape(1, d),
                    n, d, blk, norm_out=False)
    return out


# speedup vs baseline: 12.7105x; 1.0131x over previous
import functools

import jax
import jax.numpy as jnp
from jax import lax
from jax.experimental import pallas as pl
from jax.experimental.pallas import tpu as pltpu
from jax.experimental.pallas import tpu_sc as plsc

NC = 2
NS = 16
K = 50
ZR = 624

_MESH = plsc.VectorSubcoreMesh(core_axis_name="c", subcore_axis_name="s")


def _sc_degree(idx2d, zeros_n, n):
    nrows = idx2d.shape[0]
    rpw = nrows // (NC * NS)

    @functools.partial(
        pl.kernel,
        out_type=jax.ShapeDtypeStruct((2, n), jnp.float32),
        mesh=_MESH,
        compiler_params=pltpu.CompilerParams(use_tc_tiling_on_sc=False),
        scratch_types=[
            pltpu.VMEM_SHARED((n,), jnp.float32),
            pltpu.VMEM((rpw, K), jnp.int32),
            pltpu.VMEM((128,), jnp.float32),
            pltpu.VMEM((n,), jnp.float32),
            pltpu.SemaphoreType.DMA,
        ],
    )
    def deg_kernel(idx_hbm, z_hbm, out_hbm, acc, idxb, ones, stage, ssem):
        c = lax.axis_index("c")
        s = lax.axis_index("s")

        def set_ones(i, _):
            ones[pl.ds(i * 16, 16)] = jnp.full((16,), 1.0, jnp.float32)
            return 0
        lax.fori_loop(0, 8, set_ones, 0)

        @pl.when(s == 0)
        def _():
            pltpu.sync_copy(z_hbm, stage)
            pltpu.sync_copy(stage, acc)
        plsc.subcore_barrier()

        rowbase = (c * NS + s) * rpw
        pltpu.sync_copy(idx_hbm.at[pl.ds(rowbase, rpw)], idxb)

        W = 8

        def body(j, _):
            @pl.when(j >= W)
            def _():
                pltpu.make_async_copy(
                    ones.at[pl.ds(0, K)], acc.at[idxb.at[j]], ssem).wait()
            pltpu.async_copy(
                ones.at[pl.ds(0, K)], acc.at[idxb.at[j]], ssem, add=True)
            return 0
        lax.fori_loop(0, rpw, body, 0)

        def drain(j, _):
            pltpu.make_async_copy(
                ones.at[pl.ds(0, K)], acc.at[idxb.at[j]], ssem).wait()
            return 0
        lax.fori_loop(0, W, drain, 0)
        plsc.subcore_barrier()

        @pl.when(s == 0)
        def _():
            pltpu.sync_copy(acc, stage)
            pltpu.sync_copy(stage, out_hbm.at[c])

    return deg_kernel(idx2d, zeros_n)


def _sc_edge_agg(hn, src2d, dst2d, zeros_nd, n, d):
    nrows = src2d.shape[0]
    rpw = nrows // (NC * NS)
    zlast = n - (NS - 1) * ZR

    @functools.partial(
        pl.kernel,
        out_type=jax.ShapeDtypeStruct((2 * n, d), jnp.float32),
        mesh=_MESH,
        compiler_params=pltpu.CompilerParams(use_tc_tiling_on_sc=False),
        scratch_types=[
            pltpu.VMEM_SHARED((n, d), jnp.float32),
            pltpu.VMEM((rpw, K), jnp.int32),
            pltpu.VMEM((rpw, K), jnp.int32),
            pltpu.VMEM((K, d), jnp.float32),
            pltpu.VMEM((K, d), jnp.float32),
            pltpu.VMEM((K, d), jnp.float32),
            pltpu.VMEM((K, d), jnp.float32),
            pltpu.SemaphoreType.DMA,
            pltpu.SemaphoreType.DMA,
            pltpu.SemaphoreType.DMA,
            pltpu.SemaphoreType.DMA,
            pltpu.SemaphoreType.DMA,
            pltpu.SemaphoreType.DMA,
            pltpu.SemaphoreType.DMA,
            pltpu.SemaphoreType.DMA,
        ],
    )
    def edge_kernel(hn_hbm, src_hbm, dst_hbm, z_hbm, out_hbm,
                    acc, srcb, dstb, r0, r1, r2, r3,
                    g0, g1, g2, g3, s0, s1, s2, s3):
        rows = (r0, r1, r2, r3)
        gsem = (g0, g1, g2, g3)
        ssem = (s0, s1, s2, s3)
        c = lax.axis_index("c")
        s = lax.axis_index("s")

        rowbase = (c * NS + s) * rpw
        pltpu.async_copy(src_hbm.at[pl.ds(rowbase, rpw)], srcb, g0)
        pltpu.async_copy(dst_hbm.at[pl.ds(rowbase, rpw)], dstb, g1)

        @pl.when(s < NS - 1)
        def _():
            pltpu.sync_copy(z_hbm.at[pl.ds(s * ZR, ZR)],
                            acc.at[pl.ds(s * ZR, ZR)])

        @pl.when(s == NS - 1)
        def _():
            pltpu.sync_copy(z_hbm.at[pl.ds((NS - 1) * ZR, zlast)],
                            acc.at[pl.ds((NS - 1) * ZR, zlast)])

        pltpu.make_async_copy(
            src_hbm.at[pl.ds(rowbase, rpw)], srcb, g0).wait()
        pltpu.make_async_copy(
            dst_hbm.at[pl.ds(rowbase, rpw)], dstb, g1).wait()
        plsc.subcore_barrier()

        ngrp = rpw // 4
        for b in range(4):
            pltpu.async_copy(hn_hbm.at[srcb.at[b]], rows[b], gsem[b])

        def group(jj, _):
            for b in range(4):
                j = jj * 4 + b
                pltpu.make_async_copy(
                    hn_hbm.at[srcb.at[j]], rows[b], gsem[b]).wait()
                pltpu.async_copy(rows[b], acc.at[dstb.at[j]], ssem[b],
                                 add=True)

                @pl.when(jj < ngrp - 1)
                def _():
                    pltpu.make_async_copy(
                        rows[b], acc.at[dstb.at[j]], ssem[b]).wait()
                    pltpu.async_copy(
                        hn_hbm.at[srcb.at[j + 4]], rows[b], gsem[b])
            return 0
        lax.fori_loop(0, ngrp, group, 0)

        for b in range(4):
            pltpu.make_async_copy(
                rows[b], acc.at[dstb.at[rpw - 4 + b]], ssem[b]).wait()
        plsc.subcore_barrier()

        @pl.when(s < NS - 1)
        def _():
            pltpu.sync_copy(acc.at[pl.ds(s * ZR, ZR)],
                            out_hbm.at[pl.ds(c * n + s * ZR, ZR)])

        @pl.when(s == NS - 1)
        def _():
            pltpu.sync_copy(acc.at[pl.ds((NS - 1) * ZR, zlast)],
                            out_hbm.at[pl.ds(c * n + (NS - 1) * ZR, zlast)])

    return edge_kernel(hn, src2d, dst2d, zeros_nd)


def _tc_prep(h, cnt_nc, n, d, blk):
    def prep_kernel(h_ref, c_ref, hn_ref, sc_ref):
        s = lax.rsqrt(jnp.maximum(c_ref[...], 1.0))
        sc_ref[...] = s
        hn_ref[...] = h_ref[...] * s[:, 0:1]

    return pl.pallas_call(
        prep_kernel,
        grid=(n // blk,),
        in_specs=[
            pl.BlockSpec((blk, d), lambda i: (i, 0)),
            pl.BlockSpec((blk, 2), lambda i: (i, 0)),
        ],
        out_specs=[
            pl.BlockSpec((blk, d), lambda i: (i, 0)),
            pl.BlockSpec((blk, 2), lambda i: (i, 0)),
        ],
        out_shape=[
            jax.ShapeDtypeStruct((n, d), jnp.float32),
            jax.ShapeDtypeStruct((n, 2), jnp.float32),
        ],
    )(h, cnt_nc)


def _tc_layer(part, scol, W, b2d, n, d, blk, norm_out):
    nb = n // blk

    def layer_kernel(p0_ref, p1_ref, sc_ref, w_ref, b_ref, o_ref):
        agg = (p0_ref[...] + p1_ref[...]) * sc_ref[:, 1:2]
        z = jnp.dot(agg, w_ref[...], precision=lax.Precision.HIGHEST,
                    preferred_element_type=jnp.float32)
        hv = jnp.maximum(z + b_ref[...], 0.0)
        if norm_out:
            hv = hv * sc_ref[:, 0:1]
        o_ref[...] = hv

    return pl.pallas_call(
        layer_kernel,
        grid=(nb,),
        in_specs=[
            pl.BlockSpec((blk, d), lambda i: (i, 0)),
            pl.BlockSpec((blk, d), lambda i: (i + nb, 0)),
            pl.BlockSpec((blk, 2), lambda i: (i, 0)),
            pl.BlockSpec((d, d), lambda i: (0, 0)),
            pl.BlockSpec((1, d), lambda i: (0, 0)),
        ],
        out_specs=pl.BlockSpec((blk, d), lambda i: (i, 0)),
        out_shape=jax.ShapeDtypeStruct((n, d), jnp.float32),
    )(part, part, scol, W, b2d)


def kernel(h, edge_index, W1, b1, W2, b2):
    n, d = h.shape
    e = edge_index.shape[1]
    blk = 1000

    idx2d = edge_index.reshape(2 * e // K, K)
    src2d = edge_index[0].reshape(e // K, K)
    dst2d = edge_index[1].reshape(e // K, K)
    zeros_n = jnp.zeros((n,), jnp.float32)
    zeros_nd = jnp.zeros((n, d), jnp.float32)

    cnt = _sc_degree(idx2d, zeros_n, n)
    hn0, scol = _tc_prep(h, cnt.T, n, d, blk)

    part1 = _sc_edge_agg(hn0, src2d, dst2d, zeros_nd, n, d)
    h1n = _tc_layer(part1, scol, W1, b1.reshape(1, d),
                    n, d, blk, norm_out=True)

    part2 = _sc_edge_agg(h1n, src2d, dst2d, zeros_nd, n, d)
    out = _tc_layer(part2, scol, W2, b2.reshape(1, d),
                    n, d, blk, norm_out=False)
    return out


# speedup vs baseline: 14.4884x; 1.1399x over previous
import functools

import jax
import jax.numpy as jnp
from jax import lax
from jax.experimental import pallas as pl
from jax.experimental.pallas import tpu as pltpu
from jax.experimental.pallas import tpu_sc as plsc

NC = 2
NS = 16
K = 50
ZR = 624

_MESH = plsc.VectorSubcoreMesh(core_axis_name="c", subcore_axis_name="s")


def _sc_degree(idx2d, zeros_n, n):
    nrows = idx2d.shape[0]
    rpw = nrows // (NC * NS)

    @functools.partial(
        pl.kernel,
        out_type=jax.ShapeDtypeStruct((2, n), jnp.float32),
        mesh=_MESH,
        compiler_params=pltpu.CompilerParams(use_tc_tiling_on_sc=False),
        scratch_types=[
            pltpu.VMEM_SHARED((n,), jnp.float32),
            pltpu.VMEM((rpw, K), jnp.int32),
            pltpu.VMEM((128,), jnp.float32),
            pltpu.VMEM((n,), jnp.float32),
            pltpu.SemaphoreType.DMA,
        ],
    )
    def deg_kernel(idx_hbm, z_hbm, out_hbm, acc, idxb, ones, stage, ssem):
        c = lax.axis_index("c")
        s = lax.axis_index("s")

        def set_ones(i, _):
            ones[pl.ds(i * 16, 16)] = jnp.full((16,), 1.0, jnp.float32)
            return 0
        lax.fori_loop(0, 8, set_ones, 0)

        @pl.when(s == 0)
        def _():
            pltpu.sync_copy(z_hbm, stage)
            pltpu.sync_copy(stage, acc)
        plsc.subcore_barrier()

        rowbase = (c * NS + s) * rpw
        pltpu.sync_copy(idx_hbm.at[pl.ds(rowbase, rpw)], idxb)

        W = 8

        def body(j, _):
            @pl.when(j >= W)
            def _():
                pltpu.make_async_copy(
                    ones.at[pl.ds(0, K)], acc.at[idxb.at[j]], ssem).wait()
            pltpu.async_copy(
                ones.at[pl.ds(0, K)], acc.at[idxb.at[j]], ssem, add=True)
            return 0
        lax.fori_loop(0, rpw, body, 0)

        def drain(j, _):
            pltpu.make_async_copy(
                ones.at[pl.ds(0, K)], acc.at[idxb.at[j]], ssem).wait()
            return 0
        lax.fori_loop(0, W, drain, 0)
        plsc.subcore_barrier()

        @pl.when(s == 0)
        def _():
            pltpu.sync_copy(acc, stage)
            pltpu.sync_copy(stage, out_hbm.at[c])

    return deg_kernel(idx2d, zeros_n)


def _sc_edge_agg(hn, idx2d, zeros_nd, n, d):
    nrows = idx2d.shape[0] // 2
    rpw = nrows // (NC * NS)
    zlast = n - (NS - 1) * ZR

    @functools.partial(
        pl.kernel,
        out_type=jax.ShapeDtypeStruct((2 * n, d), jnp.float32),
        mesh=_MESH,
        compiler_params=pltpu.CompilerParams(use_tc_tiling_on_sc=False),
        scratch_types=[
            pltpu.VMEM_SHARED((n, d), jnp.float32),
            pltpu.VMEM((rpw, K), jnp.int32),
            pltpu.VMEM((rpw, K), jnp.int32),
            pltpu.VMEM((K, d), jnp.float32),
            pltpu.VMEM((K, d), jnp.float32),
            pltpu.VMEM((K, d), jnp.float32),
            pltpu.VMEM((K, d), jnp.float32),
            pltpu.SemaphoreType.DMA,
            pltpu.SemaphoreType.DMA,
            pltpu.SemaphoreType.DMA,
            pltpu.SemaphoreType.DMA,
            pltpu.SemaphoreType.DMA,
            pltpu.SemaphoreType.DMA,
            pltpu.SemaphoreType.DMA,
            pltpu.SemaphoreType.DMA,
        ],
    )
    def edge_kernel(hn_hbm, idx_hbm, z_hbm, out_hbm,
                    acc, srcb, dstb, r0, r1, r2, r3,
                    g0, g1, g2, g3, s0, s1, s2, s3):
        rows = (r0, r1, r2, r3)
        gsem = (g0, g1, g2, g3)
        ssem = (s0, s1, s2, s3)
        c = lax.axis_index("c")
        s = lax.axis_index("s")

        rowbase = (c * NS + s) * rpw
        pltpu.async_copy(idx_hbm.at[pl.ds(rowbase, rpw)], srcb, g0)
        pltpu.async_copy(idx_hbm.at[pl.ds(nrows + rowbase, rpw)], dstb, g1)

        @pl.when(s < NS - 1)
        def _():
            pltpu.sync_copy(z_hbm.at[pl.ds(s * ZR, ZR)],
                            acc.at[pl.ds(s * ZR, ZR)])

        @pl.when(s == NS - 1)
        def _():
            pltpu.sync_copy(z_hbm.at[pl.ds((NS - 1) * ZR, zlast)],
                            acc.at[pl.ds((NS - 1) * ZR, zlast)])

        pltpu.make_async_copy(
            idx_hbm.at[pl.ds(rowbase, rpw)], srcb, g0).wait()
        pltpu.make_async_copy(
            idx_hbm.at[pl.ds(nrows + rowbase, rpw)], dstb, g1).wait()
        plsc.subcore_barrier()

        ngrp = rpw // 4
        for b in range(4):
            pltpu.async_copy(hn_hbm.at[srcb.at[b]], rows[b], gsem[b])

        def group(jj, _):
            for b in range(4):
                j = jj * 4 + b
                pltpu.make_async_copy(
                    hn_hbm.at[srcb.at[j]], rows[b], gsem[b]).wait()
                pltpu.async_copy(rows[b], acc.at[dstb.at[j]], ssem[b],
                                 add=True)

                @pl.when(jj < ngrp - 1)
                def _():
                    pltpu.make_async_copy(
                        rows[b], acc.at[dstb.at[j]], ssem[b]).wait()
                    pltpu.async_copy(
                        hn_hbm.at[srcb.at[j + 4]], rows[b], gsem[b])
            return 0
        lax.fori_loop(0, ngrp, group, 0)

        for b in range(4):
            pltpu.make_async_copy(
                rows[b], acc.at[dstb.at[rpw - 4 + b]], ssem[b]).wait()
        plsc.subcore_barrier()

        @pl.when(s < NS - 1)
        def _():
            pltpu.sync_copy(acc.at[pl.ds(s * ZR, ZR)],
                            out_hbm.at[pl.ds(c * n + s * ZR, ZR)])

        @pl.when(s == NS - 1)
        def _():
            pltpu.sync_copy(acc.at[pl.ds((NS - 1) * ZR, zlast)],
                            out_hbm.at[pl.ds(c * n + (NS - 1) * ZR, zlast)])

    return edge_kernel(hn, idx2d, zeros_nd)


def _tc_prep(h, cnt_nc, n, d, blk):
    def prep_kernel(h_ref, c_ref, hn_ref, sc_ref):
        s = lax.rsqrt(jnp.maximum(c_ref[...], 1.0))
        sc_ref[...] = s
        hn_ref[...] = h_ref[...] * s[:, 0:1]

    return pl.pallas_call(
        prep_kernel,
        grid=(n // blk,),
        in_specs=[
            pl.BlockSpec((blk, d), lambda i: (i, 0)),
            pl.BlockSpec((blk, 2), lambda i: (i, 0)),
        ],
        out_specs=[
            pl.BlockSpec((blk, d), lambda i: (i, 0)),
            pl.BlockSpec((blk, 2), lambda i: (i, 0)),
        ],
        out_shape=[
            jax.ShapeDtypeStruct((n, d), jnp.float32),
            jax.ShapeDtypeStruct((n, 2), jnp.float32),
        ],
    )(h, cnt_nc)


def _tc_layer(part, scol, W, b2d, n, d, blk, norm_out):
    nb = n // blk

    def layer_kernel(p0_ref, p1_ref, sc_ref, w_ref, b_ref, o_ref):
        agg = (p0_ref[...] + p1_ref[...]) * sc_ref[:, 1:2]
        z = jnp.dot(agg, w_ref[...], precision=lax.Precision.DEFAULT,
                    preferred_element_type=jnp.float32)
        hv = jnp.maximum(z + b_ref[...], 0.0)
        if norm_out:
            hv = hv * sc_ref[:, 0:1]
        o_ref[...] = hv

    return pl.pallas_call(
        layer_kernel,
        grid=(nb,),
        in_specs=[
            pl.BlockSpec((blk, d), lambda i: (i, 0)),
            pl.BlockSpec((blk, d), lambda i: (i + nb, 0)),
            pl.BlockSpec((blk, 2), lambda i: (i, 0)),
            pl.BlockSpec((d, d), lambda i: (0, 0)),
            pl.BlockSpec((1, d), lambda i: (0, 0)),
        ],
        out_specs=pl.BlockSpec((blk, d), lambda i: (i, 0)),
        out_shape=jax.ShapeDtypeStruct((n, d), jnp.float32),
    )(part, part, scol, W, b2d)


def kernel(h, edge_index, W1, b1, W2, b2):
    n, d = h.shape
    e = edge_index.shape[1]
    blk = 2000

    idx2d = edge_index.reshape(2 * e // K, K)
    zeros_n = jnp.zeros((n,), jnp.float32)
    zeros_nd = jnp.zeros((n, d), jnp.float32)

    cnt = _sc_degree(idx2d, zeros_n, n)
    hn0, scol = _tc_prep(h, cnt.T, n, d, blk)

    part1 = _sc_edge_agg(hn0, idx2d, zeros_nd, n, d)
    h1n = _tc_layer(part1, scol, W1, b1.reshape(1, d),
                    n, d, blk, norm_out=True)

    part2 = _sc_edge_agg(h1n, idx2d, zeros_nd, n, d)
    out = _tc_layer(part2, scol, W2, b2.reshape(1, d),
                    n, d, blk, norm_out=False)
    return out


# speedup vs baseline: 14.9519x; 1.0320x over previous
import functools

import jax
import jax.numpy as jnp
from jax import lax
from jax.experimental import pallas as pl
from jax.experimental.pallas import tpu as pltpu
from jax.experimental.pallas import tpu_sc as plsc

NC = 2
NS = 16
K = 50
ZR = 624

_MESH = plsc.VectorSubcoreMesh(core_axis_name="c", subcore_axis_name="s")


def _sc_degree(idx2d, n):
    nrows = idx2d.shape[0]
    rpw = nrows // (NC * NS)

    @functools.partial(
        pl.kernel,
        out_type=jax.ShapeDtypeStruct((2, n), jnp.float32),
        mesh=_MESH,
        compiler_params=pltpu.CompilerParams(use_tc_tiling_on_sc=False),
        scratch_types=[
            pltpu.VMEM_SHARED((n,), jnp.float32),
            pltpu.VMEM((rpw, K), jnp.int32),
            pltpu.VMEM((128,), jnp.float32),
            pltpu.VMEM((n,), jnp.float32),
            pltpu.SemaphoreType.DMA,
            pltpu.SemaphoreType.DMA,
        ],
    )
    def deg_kernel(idx_hbm, out_hbm, acc, idxb, ones, stage, ssem, lsem):
        c = lax.axis_index("c")
        s = lax.axis_index("s")

        rowbase = (c * NS + s) * rpw
        pltpu.async_copy(idx_hbm.at[pl.ds(rowbase, rpw)], idxb, lsem)

        def set_ones(i, _):
            ones[pl.ds(i * 16, 16)] = jnp.full((16,), 1.0, jnp.float32)
            return 0
        lax.fori_loop(0, 8, set_ones, 0)

        @pl.when(s == 0)
        def _():
            def zstage(i, _):
                stage[pl.ds(i * 16, 16)] = jnp.zeros((16,), jnp.float32)
                return 0
            lax.fori_loop(0, n // 16, zstage, 0)
            pltpu.sync_copy(stage, acc)

        pltpu.make_async_copy(
            idx_hbm.at[pl.ds(rowbase, rpw)], idxb, lsem).wait()
        plsc.subcore_barrier()

        W = 8

        def body(j, _):
            @pl.when(j >= W)
            def _():
                pltpu.make_async_copy(
                    ones.at[pl.ds(0, K)], acc.at[idxb.at[j]], ssem).wait()
            pltpu.async_copy(
                ones.at[pl.ds(0, K)], acc.at[idxb.at[j]], ssem, add=True)
            return 0
        lax.fori_loop(0, rpw, body, 0)

        def drain(j, _):
            pltpu.make_async_copy(
                ones.at[pl.ds(0, K)], acc.at[idxb.at[j]], ssem).wait()
            return 0
        lax.fori_loop(0, W, drain, 0)
        plsc.subcore_barrier()

        @pl.when(s == 0)
        def _():
            pltpu.sync_copy(acc, stage)
            pltpu.sync_copy(stage, out_hbm.at[c])

    return deg_kernel(idx2d)


def _sc_edge_agg(hn, idx2d, n, d):
    nrows = idx2d.shape[0] // 2
    rpw = nrows // (NC * NS)
    zlast = n - (NS - 1) * ZR

    @functools.partial(
        pl.kernel,
        out_type=jax.ShapeDtypeStruct((2 * n, d), jnp.float32),
        mesh=_MESH,
        compiler_params=pltpu.CompilerParams(use_tc_tiling_on_sc=False),
        scratch_types=[
            pltpu.VMEM_SHARED((n, d), jnp.float32),
            pltpu.VMEM((rpw, K), jnp.int32),
            pltpu.VMEM((rpw, K), jnp.int32),
            pltpu.VMEM((K, d), jnp.float32),
            pltpu.VMEM((K, d), jnp.float32),
            pltpu.VMEM((K, d), jnp.float32),
            pltpu.VMEM((K, d), jnp.float32),
            pltpu.SemaphoreType.DMA,
            pltpu.SemaphoreType.DMA,
            pltpu.SemaphoreType.DMA,
            pltpu.SemaphoreType.DMA,
            pltpu.SemaphoreType.DMA,
            pltpu.SemaphoreType.DMA,
            pltpu.SemaphoreType.DMA,
            pltpu.SemaphoreType.DMA,
        ],
    )
    def edge_kernel(hn_hbm, idx_hbm, out_hbm,
                    acc, srcb, dstb, r0, r1, r2, r3,
                    g0, g1, g2, g3, s0, s1, s2, s3):
        rows = (r0, r1, r2, r3)
        gsem = (g0, g1, g2, g3)
        ssem = (s0, s1, s2, s3)
        c = lax.axis_index("c")
        s = lax.axis_index("s")

        rowbase = (c * NS + s) * rpw
        pltpu.async_copy(idx_hbm.at[pl.ds(rowbase, rpw)], srcb, g0)
        pltpu.async_copy(idx_hbm.at[pl.ds(nrows + rowbase, rpw)], dstb, g1)

        def zrow(i, _):
            for j in range(d // 16):
                r0[i, pl.ds(j * 16, 16)] = jnp.zeros((16,), jnp.float32)
            return 0
        lax.fori_loop(0, K, zrow, 0)

        nz = ZR // 48

        def zcopy(k, _):
            pltpu.sync_copy(r0.at[pl.ds(0, 48)],
                            acc.at[pl.ds(s * ZR + k * 48, 48)])
            return 0
        lax.fori_loop(0, nz, zcopy, 0)

        @pl.when(s == NS - 1)
        def _():
            pltpu.sync_copy(
                r0.at[pl.ds(0, zlast - ZR)],
                acc.at[pl.ds((NS - 1) * ZR + nz * 48, zlast - ZR)])

        pltpu.make_async_copy(
            idx_hbm.at[pl.ds(rowbase, rpw)], srcb, g0).wait()
        pltpu.make_async_copy(
            idx_hbm.at[pl.ds(nrows + rowbase, rpw)], dstb, g1).wait()
        plsc.subcore_barrier()

        ngrp = rpw // 4
        for b in range(4):
            pltpu.async_copy(hn_hbm.at[srcb.at[b]], rows[b], gsem[b])

        def group(jj, _):
            for b in range(4):
                j = jj * 4 + b
                pltpu.make_async_copy(
                    hn_hbm.at[srcb.at[j]], rows[b], gsem[b]).wait()
                pltpu.async_copy(rows[b], acc.at[dstb.at[j]], ssem[b],
                                 add=True)

                @pl.when(jj < ngrp - 1)
                def _():
                    pltpu.make_async_copy(
                        rows[b], acc.at[dstb.at[j]], ssem[b]).wait()
                    pltpu.async_copy(
                        hn_hbm.at[srcb.at[j + 4]], rows[b], gsem[b])
            return 0
        lax.fori_loop(0, ngrp, group, 0)

        for b in range(4):
            pltpu.make_async_copy(
                rows[b], acc.at[dstb.at[rpw - 4 + b]], ssem[b]).wait()
        plsc.subcore_barrier()

        @pl.when(s < NS - 1)
        def _():
            pltpu.sync_copy(acc.at[pl.ds(s * ZR, ZR)],
                            out_hbm.at[pl.ds(c * n + s * ZR, ZR)])

        @pl.when(s == NS - 1)
        def _():
            pltpu.sync_copy(acc.at[pl.ds((NS - 1) * ZR, zlast)],
                            out_hbm.at[pl.ds(c * n + (NS - 1) * ZR, zlast)])

    return edge_kernel(hn, idx2d)


def _tc_prep(h, cnt_nc, n, d, blk):
    def prep_kernel(h_ref, c_ref, hn_ref, sc_ref):
        s = lax.rsqrt(jnp.maximum(c_ref[...], 1.0))
        sc_ref[...] = s
        hn_ref[...] = h_ref[...] * s[:, 0:1]

    return pl.pallas_call(
        prep_kernel,
        grid=(n // blk,),
        in_specs=[
            pl.BlockSpec((blk, d), lambda i: (i, 0)),
            pl.BlockSpec((blk, 2), lambda i: (i, 0)),
        ],
        out_specs=[
            pl.BlockSpec((blk, d), lambda i: (i, 0)),
            pl.BlockSpec((blk, 2), lambda i: (i, 0)),
        ],
        out_shape=[
            jax.ShapeDtypeStruct((n, d), jnp.float32),
            jax.ShapeDtypeStruct((n, 2), jnp.float32),
        ],
    )(h, cnt_nc)


def _tc_layer(part, scol, W, b2d, n, d, blk, norm_out):
    nb = n // blk

    def layer_kernel(p0_ref, p1_ref, sc_ref, w_ref, b_ref, o_ref):
        agg = (p0_ref[...] + p1_ref[...]) * sc_ref[:, 1:2]
        z = jnp.dot(agg, w_ref[...], precision=lax.Precision.DEFAULT,
                    preferred_element_type=jnp.float32)
        hv = jnp.maximum(z + b_ref[...], 0.0)
        if norm_out:
            hv = hv * sc_ref[:, 0:1]
        o_ref[...] = hv

    return pl.pallas_call(
        layer_kernel,
        grid=(nb,),
        in_specs=[
            pl.BlockSpec((blk, d), lambda i: (i, 0)),
            pl.BlockSpec((blk, d), lambda i: (i + nb, 0)),
            pl.BlockSpec((blk, 2), lambda i: (i, 0)),
            pl.BlockSpec((d, d), lambda i: (0, 0)),
            pl.BlockSpec((1, d), lambda i: (0, 0)),
        ],
        out_specs=pl.BlockSpec((blk, d), lambda i: (i, 0)),
        out_shape=jax.ShapeDtypeStruct((n, d), jnp.float32),
    )(part, part, scol, W, b2d)


def kernel(h, edge_index, W1, b1, W2, b2):
    n, d = h.shape
    e = edge_index.shape[1]
    blk = 2000

    idx2d = edge_index.reshape(2 * e // K, K)

    cnt = _sc_degree(idx2d, n)
    hn0, scol = _tc_prep(h, cnt.T, n, d, blk)

    part1 = _sc_edge_agg(hn0, idx2d, n, d)
    h1n = _tc_layer(part1, scol, W1, b1.reshape(1, d),
                    n, d, blk, norm_out=True)

    part2 = _sc_edge_agg(h1n, idx2d, n, d)
    out = _tc_layer(part2, scol, W2, b2.reshape(1, d),
                    n, d, blk, norm_out=False)
    return out


# speedup vs baseline: 14.9541x; 1.0001x over previous
import functools

import jax
import jax.numpy as jnp
from jax import lax
from jax.experimental import pallas as pl
from jax.experimental.pallas import tpu as pltpu
from jax.experimental.pallas import tpu_sc as plsc

NC = 2
NS = 16
K = 50
NBUF = 4
ZR = 624

_MESH = plsc.VectorSubcoreMesh(core_axis_name="c", subcore_axis_name="s")


def _sc_degree(idx2d, n):
    nrows = idx2d.shape[0]
    rpw = nrows // (NC * NS)

    @functools.partial(
        pl.kernel,
        out_type=jax.ShapeDtypeStruct((2, n), jnp.float32),
        mesh=_MESH,
        compiler_params=pltpu.CompilerParams(use_tc_tiling_on_sc=False),
        scratch_types=[
            pltpu.VMEM_SHARED((n,), jnp.float32),
            pltpu.VMEM((rpw, K), jnp.int32),
            pltpu.VMEM((128,), jnp.float32),
            pltpu.VMEM((n,), jnp.float32),
            pltpu.SemaphoreType.DMA,
            pltpu.SemaphoreType.DMA,
        ],
    )
    def deg_kernel(idx_hbm, out_hbm, acc, idxb, ones, stage, ssem, lsem):
        c = lax.axis_index("c")
        s = lax.axis_index("s")

        rowbase = (c * NS + s) * rpw
        pltpu.async_copy(idx_hbm.at[pl.ds(rowbase, rpw)], idxb, lsem)

        def set_ones(i, _):
            ones[pl.ds(i * 16, 16)] = jnp.full((16,), 1.0, jnp.float32)
            return 0
        lax.fori_loop(0, 8, set_ones, 0)

        @pl.when(s == 0)
        def _():
            def zstage(i, _):
                stage[pl.ds(i * 16, 16)] = jnp.zeros((16,), jnp.float32)
                return 0
            lax.fori_loop(0, n // 16, zstage, 0)
            pltpu.sync_copy(stage, acc)

        pltpu.make_async_copy(
            idx_hbm.at[pl.ds(rowbase, rpw)], idxb, lsem).wait()
        plsc.subcore_barrier()

        W = 8

        def body(j, _):
            @pl.when(j >= W)
            def _():
                pltpu.make_async_copy(
                    ones.at[pl.ds(0, K)], acc.at[idxb.at[j]], ssem).wait()
            pltpu.async_copy(
                ones.at[pl.ds(0, K)], acc.at[idxb.at[j]], ssem, add=True)
            return 0
        lax.fori_loop(0, rpw, body, 0)

        def drain(j, _):
            pltpu.make_async_copy(
                ones.at[pl.ds(0, K)], acc.at[idxb.at[j]], ssem).wait()
            return 0
        lax.fori_loop(0, W, drain, 0)
        plsc.subcore_barrier()

        @pl.when(s == 0)
        def _():
            pltpu.sync_copy(acc, stage)
            pltpu.sync_copy(stage, out_hbm.at[c])

    return deg_kernel(idx2d)


def _sc_edge_agg(hn, idx2d, n, d):
    nrows = idx2d.shape[0] // 2
    rpw = nrows // (NC * NS)
    zlast = n - (NS - 1) * ZR

    @functools.partial(
        pl.kernel,
        out_type=jax.ShapeDtypeStruct((2 * n, d), jnp.float32),
        mesh=_MESH,
        compiler_params=pltpu.CompilerParams(use_tc_tiling_on_sc=False),
        scratch_types=[
            pltpu.VMEM_SHARED((n, d), jnp.float32),
            pltpu.VMEM((rpw, K), jnp.int32),
            pltpu.VMEM((rpw, K), jnp.int32),
            pltpu.VMEM((K, d), jnp.float32),
            pltpu.VMEM((K, d), jnp.float32),
            pltpu.VMEM((K, d), jnp.float32),
            pltpu.VMEM((K, d), jnp.float32),
            pltpu.SemaphoreType.DMA,
            pltpu.SemaphoreType.DMA,
            pltpu.SemaphoreType.DMA,
            pltpu.SemaphoreType.DMA,
            pltpu.SemaphoreType.DMA,
            pltpu.SemaphoreType.DMA,
            pltpu.SemaphoreType.DMA,
            pltpu.SemaphoreType.DMA,
        ],
    )
    def edge_kernel(hn_hbm, idx_hbm, out_hbm,
                    acc, srcb, dstb, r0, r1, r2, r3,
                    g0, g1, g2, g3, s0, s1, s2, s3):
        rows = (r0, r1, r2, r3)
        gsem = (g0, g1, g2, g3)
        ssem = (s0, s1, s2, s3)
        c = lax.axis_index("c")
        s = lax.axis_index("s")

        rowbase = (c * NS + s) * rpw
        pltpu.async_copy(idx_hbm.at[pl.ds(rowbase, rpw)], srcb, g0)
        pltpu.async_copy(idx_hbm.at[pl.ds(nrows + rowbase, rpw)], dstb, g1)

        def zrow(i, _):
            for j in range(d // 16):
                r0[i, pl.ds(j * 16, 16)] = jnp.zeros((16,), jnp.float32)
            return 0
        lax.fori_loop(0, K, zrow, 0)

        nz = ZR // 48

        def zcopy(k, _):
            pltpu.sync_copy(r0.at[pl.ds(0, 48)],
                            acc.at[pl.ds(s * ZR + k * 48, 48)])
            return 0
        lax.fori_loop(0, nz, zcopy, 0)

        @pl.when(s == NS - 1)
        def _():
            pltpu.sync_copy(
                r0.at[pl.ds(0, zlast - ZR)],
                acc.at[pl.ds((NS - 1) * ZR + nz * 48, zlast - ZR)])

        pltpu.make_async_copy(
            idx_hbm.at[pl.ds(rowbase, rpw)], srcb, g0).wait()
        pltpu.make_async_copy(
            idx_hbm.at[pl.ds(nrows + rowbase, rpw)], dstb, g1).wait()
        plsc.subcore_barrier()

        ngrp = rpw // NBUF
        for b in range(NBUF):
            pltpu.async_copy(hn_hbm.at[srcb.at[b]], rows[b], gsem[b])

        def group(jj, _):
            for b in range(NBUF):
                j = jj * NBUF + b
                pltpu.make_async_copy(
                    hn_hbm.at[srcb.at[j]], rows[b], gsem[b]).wait()
                pltpu.async_copy(rows[b], acc.at[dstb.at[j]], ssem[b],
                                 add=True)

                @pl.when(jj < ngrp - 1)
                def _():
                    pltpu.make_async_copy(
                        rows[b], acc.at[dstb.at[j]], ssem[b]).wait()
                    pltpu.async_copy(
                        hn_hbm.at[srcb.at[j + NBUF]], rows[b], gsem[b])
            return 0
        lax.fori_loop(0, ngrp, group, 0)

        for b in range(NBUF):
            pltpu.make_async_copy(
                rows[b], acc.at[dstb.at[rpw - NBUF + b]], ssem[b]).wait()
        plsc.subcore_barrier()

        @pl.when(s < NS - 1)
        def _():
            pltpu.sync_copy(acc.at[pl.ds(s * ZR, ZR)],
                            out_hbm.at[pl.ds(c * n + s * ZR, ZR)])

        @pl.when(s == NS - 1)
        def _():
            pltpu.sync_copy(acc.at[pl.ds((NS - 1) * ZR, zlast)],
                            out_hbm.at[pl.ds(c * n + (NS - 1) * ZR, zlast)])

    return edge_kernel(hn, idx2d)


def _tc_prep(h, cnt_nc, n, d, blk):
    def prep_kernel(h_ref, c_ref, hn_ref, sc_ref):
        s = lax.rsqrt(jnp.maximum(c_ref[...], 1.0))
        sc_ref[...] = s
        hn_ref[...] = h_ref[...] * s[:, 0:1]

    return pl.pallas_call(
        prep_kernel,
        grid=(n // blk,),
        in_specs=[
            pl.BlockSpec((blk, d), lambda i: (i, 0)),
            pl.BlockSpec((blk, 2), lambda i: (i, 0)),
        ],
        out_specs=[
            pl.BlockSpec((blk, d), lambda i: (i, 0)),
            pl.BlockSpec((blk, 2), lambda i: (i, 0)),
        ],
        out_shape=[
            jax.ShapeDtypeStruct((n, d), jnp.float32),
            jax.ShapeDtypeStruct((n, 2), jnp.float32),
        ],
    )(h, cnt_nc)


def _tc_layer(part, scol, W, b2d, n, d, blk, norm_out):
    nb = n // blk

    def layer_kernel(p0_ref, p1_ref, sc_ref, w_ref, b_ref, o_ref):
        agg = (p0_ref[...] + p1_ref[...]) * sc_ref[:, 1:2]
        z = jnp.dot(agg, w_ref[...], precision=lax.Precision.DEFAULT,
                    preferred_element_type=jnp.float32)
        hv = jnp.maximum(z + b_ref[...], 0.0)
        if norm_out:
            hv = hv * sc_ref[:, 0:1]
        o_ref[...] = hv

    return pl.pallas_call(
        layer_kernel,
        grid=(nb,),
        in_specs=[
            pl.BlockSpec((blk, d), lambda i: (i, 0)),
            pl.BlockSpec((blk, d), lambda i: (i + nb, 0)),
            pl.BlockSpec((blk, 2), lambda i: (i, 0)),
            pl.BlockSpec((d, d), lambda i: (0, 0)),
            pl.BlockSpec((1, d), lambda i: (0, 0)),
        ],
        out_specs=pl.BlockSpec((blk, d), lambda i: (i, 0)),
        out_shape=jax.ShapeDtypeStruct((n, d), jnp.float32),
    )(part, part, scol, W, b2d)


def kernel(h, edge_index, W1, b1, W2, b2):
    n, d = h.shape
    e = edge_index.shape[1]
    blk = 2000

    idx2d = edge_index.reshape(2 * e // K, K)

    cnt = _sc_degree(idx2d, n)
    hn0, scol = _tc_prep(h, cnt.T, n, d, blk)

    part1 = _sc_edge_agg(hn0, idx2d, n, d)
    h1n = _tc_layer(part1, scol, W1, b1.reshape(1, d),
                    n, d, blk, norm_out=True)

    part2 = _sc_edge_agg(h1n, idx2d, n, d)
    out = _tc_layer(part2, scol, W2, b2.reshape(1, d),
                    n, d, blk, norm_out=False)
    return out
